# baseline (speedup 1.0000x reference)
import functools

import jax
import jax.numpy as jnp
from jax.experimental import pallas as pl
from jax.experimental.pallas import tpu as pltpu

_K = 1024
_D = 256
_N = 18432
_ROWS = 1152


def _vq_body(x_ref, cbt_ref, out_ref):
    x = x_ref[...]
    cbt = cbt_ref[...]
    mm = jnp.dot(x, cbt, preferred_element_type=jnp.float32)
    c_sqr = jnp.sum(cbt * cbt, axis=0, keepdims=True)
    x_sqr = jnp.sum(x * x, axis=1, keepdims=True)
    dist = (c_sqr + x_sqr) - 2.0 * mm
    m = jnp.min(dist, axis=1, keepdims=True)
    iota = jax.lax.broadcasted_iota(jnp.int32, (_ROWS, _K), 1)
    cand = jnp.where(dist == m, iota, _K)
    out_ref[0, 0, :] = jnp.min(cand, axis=1)


@jax.jit
def kernel(z_e_x, codebook):
    b, c, h, w = z_e_x.shape
    x = jnp.transpose(z_e_x, (0, 2, 3, 1)).reshape(-1, c)
    cbt = codebook.T
    n_blocks = _N // _ROWS
    out = pl.pallas_call(
        _vq_body,
        grid=(n_blocks,),
        in_specs=[
            pl.BlockSpec((_ROWS, _D), lambda i: (i, 0)),
            pl.BlockSpec((_D, _K), lambda i: (0, 0)),
        ],
        out_specs=pl.BlockSpec((1, 1, _ROWS), lambda i: (i, 0, 0)),
        out_shape=jax.ShapeDtypeStruct((n_blocks, 1, _ROWS), jnp.int32),
    )(x, cbt)
    return out.reshape(b, h, w)


# speedup vs baseline: 1.0623x; 1.0623x over previous
import jax
import jax.numpy as jnp
from jax.experimental import pallas as pl

_K = 1024
_D = 256
_P = 576
_BPS = 2


def _vq_body(z_ref, cb2_ref, out_ref):
    cb2 = cb2_ref[...]
    z = z_ref[...]
    zc = jnp.concatenate([z[i] for i in range(_BPS)], axis=1)
    mm2 = jnp.dot(cb2, zc, preferred_element_type=jnp.float32)
    c_sqr = 0.25 * jnp.sum(cb2 * cb2, axis=1, keepdims=True)
    x_sqr = jnp.sum(zc * zc, axis=0, keepdims=True)
    dist = (c_sqr + x_sqr) + mm2
    m = jnp.min(dist, axis=0, keepdims=True)
    iota = jax.lax.broadcasted_iota(jnp.int32, (_K, 1), 0).astype(jnp.float32)
    cand = jnp.where(dist == m, iota, float(_K))
    out_ref[0, 0, :] = jnp.min(cand, axis=0).astype(jnp.int32)


@jax.jit
def kernel(z_e_x, codebook):
    b, c, h, w = z_e_x.shape
    z = z_e_x.reshape(b, c, h * w)
    cb2 = -2.0 * codebook
    n_blocks = b // _BPS
    out = pl.pallas_call(
        _vq_body,
        grid=(n_blocks,),
        in_specs=[
            pl.BlockSpec((_BPS, _D, _P), lambda i: (i, 0, 0)),
            pl.BlockSpec((_K, _D), lambda i: (0, 0)),
        ],
        out_specs=pl.BlockSpec((1, 1, _BPS * _P), lambda i: (i, 0, 0)),
        out_shape=jax.ShapeDtypeStruct((n_blocks, 1, _BPS * _P), jnp.int32),
    )(z, cb2)
    return out.reshape(b, h, w)
